# baseline (speedup 1.0000x reference)
import jax
import jax.numpy as jnp
from jax import lax
from jax.experimental import pallas as pl
from jax.experimental.pallas import tpu as pltpu
from jax.experimental.pallas import tpu_sc as plsc

NC = 2
NS = 16
L = 16
NW = NC * NS
CH = 128
CH3 = 3 * CH
NB = 2


def _rsqrt(x):
    i = lax.bitcast_convert_type(x, jnp.int32)
    i = jnp.int32(0x5F3759DF) - (i >> 1)
    y = lax.bitcast_convert_type(i, jnp.float32)
    y = y * (1.5 - 0.5 * x * y * y)
    y = y * (1.5 - 0.5 * x * y * y)
    return y


def _cross(a, b):
    return (a[1] * b[2] - a[2] * b[1],
            a[2] * b[0] - a[0] * b[2],
            a[0] * b[1] - a[1] * b[0])


def _norm3(v):
    r = _rsqrt(v[0] * v[0] + v[1] * v[1] + v[2] * v[2])
    return v[0] * r, v[1] * r, v[2] * r


def _frame(a, b, c):
    d = (b[0] - a[0], b[1] - a[1], b[2] - a[2])
    e = (c[0] - a[0], c[1] - a[1], c[2] - a[2])
    n = _cross(d, e)
    X = _norm3(_cross(d, n))
    Z = _norm3(d)
    Y = _cross(Z, X)
    return X, Y, Z


def _quat_w(ca, cb, cc, ma, mb, mc):
    Xc, Yc, Zc = _frame(ca, cb, cc)
    Xd, Yd, Zd = _frame(ma, mb, mc)

    def m(i, j):
        return Xd[i] * Xc[j] + Yd[i] * Yc[j] + Zd[i] * Zc[j]

    m00, m01, m02 = m(0, 0), m(0, 1), m(0, 2)
    m10, m11, m12 = m(1, 0), m(1, 1), m(1, 2)
    m20, m21, m22 = m(2, 0), m(2, 1), m(2, 2)

    def sqrtpos(t):
        pos = t > 0.0
        return jnp.where(pos, t * _rsqrt(jnp.where(pos, t, 1.0)), 0.0)

    q0 = sqrtpos(1.0 + m00 + m11 + m22)
    q1 = sqrtpos(1.0 + m00 - m11 - m22)
    q2 = sqrtpos(1.0 - m00 + m11 - m22)
    q3 = sqrtpos(1.0 - m00 - m11 + m22)

    r0 = (q0 * q0, m21 - m12, m02 - m20, m10 - m01)
    r1 = (m21 - m12, q1 * q1, m10 + m01, m02 + m20)
    r2 = (m02 - m20, m10 + m01, q2 * q2, m12 + m21)
    r3 = (m10 - m01, m20 + m02, m21 + m12, q3 * q3)

    d0 = 2.0 * jnp.maximum(q0, 0.1)
    d1 = 2.0 * jnp.maximum(q1, 0.1)
    d2 = 2.0 * jnp.maximum(q2, 0.1)
    d3 = 2.0 * jnp.maximum(q3, 0.1)

    qm = jnp.maximum(jnp.maximum(q0, q1), jnp.maximum(q2, q3))
    s0, s1, s2 = q0 == qm, q1 == qm, q2 == qm

    u = (cc[0] - cb[0], cc[1] - cb[1], cc[2] - cb[2])
    v = (ca[0] - cb[0], ca[1] - cb[1], ca[2] - cb[2])
    f = _cross(u, v)
    ss = f[0] * f[0] + f[1] * f[1] + f[2] * f[2]
    area = 0.5 * ss * _rsqrt(jnp.maximum(ss, 1e-38))

    out = []
    for k in range(4):
        q = jnp.where(s0, r0[k] / d0,
            jnp.where(s1, r1[k] / d1,
            jnp.where(s2, r2[k] / d2, r3[k] / d3)))
        out.append(area * q)
    return out


def _build_face_kernel(Vp, FW, acc_tile):
    mesh = plsc.VectorSubcoreMesh(core_axis_name="c", subcore_axis_name="s",
                                  num_cores=NC)
    cw = FW // CH

    @pl.kernel(
        out_type=jax.ShapeDtypeStruct((NC * 4 * Vp,), jnp.float32),
        mesh=mesh,
        scratch_types=(
            [pltpu.VMEM((CH3,), jnp.int32) for _ in range(NB)]
            + [pltpu.VMEM((CH3,), jnp.float32) for _ in range(6 * NB)]
            + [pltpu.VMEM((CH3,), jnp.float32) for _ in range(4)]
            + [pltpu.VMEM((acc_tile,), jnp.float32)]
            + [pltpu.VMEM_SHARED((Vp,), jnp.float32) for _ in range(4)]
            + [pltpu.SemaphoreType.DMA for _ in range(NB)]
        ),
    )
    def face_kernel(cx, cy, cz, mx, my, mz, f_all, out_hbm, *scr):
        ivs = [scr[b] for b in range(NB)]
        o = NB
        gbs = [scr[o + 6 * b:o + 6 * b + 6] for b in range(NB)]
        o += 6 * NB
        wvs = scr[o:o + 4]
        stage = scr[o + 4]
        accs = scr[o + 5:o + 9]
        sems = scr[o + 9:o + 9 + NB]

        c = lax.axis_index("c")
        s = lax.axis_index("s")
        wid = s * NC + c
        tables = (cx, cy, cz, mx, my, mz)

        def zero_grp(g, carry):
            stage[pl.ds(pl.multiple_of(g * L, 8), L)] = jnp.zeros(
                (L,), jnp.float32)
            return carry

        lax.fori_loop(0, acc_tile // L, zero_grp, 0)
        zb = pl.multiple_of(s * acc_tile, 8)
        for k in range(4):
            pltpu.sync_copy(stage, accs[k].at[pl.ds(zb, acc_tile)])
        plsc.subcore_barrier()

        def load_idx(iv, chunk):
            base = pl.multiple_of((wid * cw + chunk) * CH3, 8)
            pltpu.sync_copy(f_all.at[pl.ds(base, CH3)], iv)

        def issue_gathers(iv, gb, sem):
            for ti in range(6):
                pltpu.async_copy(tables[ti].at[iv], gb[ti], sem)

        def drain_gathers(iv, gb, sem):
            for ti in range(6):
                pltpu.make_async_copy(tables[ti].at[iv], gb[ti], sem).wait()

        for b in range(NB):
            load_idx(ivs[b], b)
            issue_gathers(ivs[b], gbs[b], sems[b])

        def pair(t, carry):
            for b in range(NB):
                iv, gb, sem = ivs[b], gbs[b], sems[b]
                chunk = NB * t + b
                drain_gathers(iv, gb, sem)

                def group(g, carry2):
                    sla = pl.ds(pl.multiple_of(g * L, 8), L)
                    slb = pl.ds(pl.multiple_of(CH + g * L, 8), L)
                    slc = pl.ds(pl.multiple_of(2 * CH + g * L, 8), L)
                    w = _quat_w(
                        (gb[0][sla], gb[1][sla], gb[2][sla]),
                        (gb[0][slb], gb[1][slb], gb[2][slb]),
                        (gb[0][slc], gb[1][slc], gb[2][slc]),
                        (gb[3][sla], gb[4][sla], gb[5][sla]),
                        (gb[3][slb], gb[4][slb], gb[5][slb]),
                        (gb[3][slc], gb[4][slc], gb[5][slc]))
                    for k in range(4):
                        wvs[k][sla] = w[k]
                        wvs[k][slb] = w[k]
                        wvs[k][slc] = w[k]
                    return carry2

                lax.fori_loop(0, CH // L, group, 0)

                for k in range(4):
                    pltpu.sync_copy(wvs[k], accs[k].at[iv], add=True)

                load_idx(iv, chunk + NB)
                issue_gathers(iv, gb, sem)
            return carry

        lax.fori_loop(0, cw // NB, pair, 0)
        for b in range(NB):
            drain_gathers(ivs[b], gbs[b], sems[b])
        plsc.subcore_barrier()

        for k in range(4):
            ob = pl.multiple_of((c * 4 + k) * Vp + zb, 8)
            pltpu.sync_copy(accs[k].at[pl.ds(zb, acc_tile)], stage)
            pltpu.sync_copy(stage, out_hbm.at[pl.ds(ob, acc_tile)])

    return face_kernel


def _build_combine_kernel(Vp, VW):
    mesh = plsc.VectorSubcoreMesh(core_axis_name="c", subcore_axis_name="s",
                                  num_cores=NC)

    @pl.kernel(
        out_type=jax.ShapeDtypeStruct((4 * Vp,), jnp.float32),
        mesh=mesh,
        scratch_types=(
            [pltpu.VMEM((VW,), jnp.float32) for _ in range(12)]
        ),
    )
    def combine_kernel(parts_hbm, out_hbm,
                       p0, p1, p2, p3, q0, q1, q2, q3, o0, o1, o2, o3):
        c = lax.axis_index("c")
        s = lax.axis_index("s")
        wid = s * NC + c
        base = pl.multiple_of(wid * VW, 8)
        pv = (p0, p1, p2, p3)
        qv = (q0, q1, q2, q3)
        ov = (o0, o1, o2, o3)
        for k in range(4):
            pb = pl.multiple_of(k * Vp + base, 8)
            qb = pl.multiple_of((4 + k) * Vp + base, 8)
            pltpu.sync_copy(parts_hbm.at[pl.ds(pb, VW)], pv[k])
            pltpu.sync_copy(parts_hbm.at[pl.ds(qb, VW)], qv[k])

        def grp(g, carry):
            sl = pl.ds(pl.multiple_of(g * L, 8), L)
            sv = [pv[k][sl] + qv[k][sl] for k in range(4)]
            ss = sv[0] * sv[0] + sv[1] * sv[1] + sv[2] * sv[2] + sv[3] * sv[3]
            r = _rsqrt(jnp.maximum(ss, 1e-38))
            nrm = ss * r
            inv = jnp.where(nrm > 1e-6, r, 1e6)
            for k in range(4):
                ov[k][sl] = sv[k] * inv
            return carry

        lax.fori_loop(0, VW // L, grp, 0)
        for k in range(4):
            ob = pl.multiple_of(k * Vp + base, 8)
            pltpu.sync_copy(ov[k], out_hbm.at[pl.ds(ob, VW)])

    return combine_kernel


def kernel(mesh_verts, cano_verts, cano_faces):
    V = cano_verts.shape[0]
    F = cano_faces.shape[0]

    FW = -(-F // (NW * NB * CH)) * NB * CH
    n_chunks = (NW * FW) // CH + NB
    F_pad = n_chunks * CH
    VW = -(-V // (NW * L)) * L
    Vp = NW * VW
    acc_tile = Vp // NS

    cano = cano_verts.astype(jnp.float32)
    dmesh = mesh_verts.astype(jnp.float32)
    faces = jnp.pad(cano_faces.astype(jnp.int32), ((0, F_pad - F), (0, 0)))
    f_all = faces.reshape(n_chunks, CH, 3).transpose(0, 2, 1).reshape(-1)

    parts = _build_face_kernel(Vp, FW, acc_tile)(
        cano[:, 0], cano[:, 1], cano[:, 2],
        dmesh[:, 0], dmesh[:, 1], dmesh[:, 2],
        f_all)
    out = _build_combine_kernel(Vp, VW)(parts)
    return out.reshape(4, Vp).T[:V]


# speedup vs baseline: 1.0745x; 1.0745x over previous
import jax
import jax.numpy as jnp
from jax import lax
from jax.experimental import pallas as pl
from jax.experimental.pallas import tpu as pltpu
from jax.experimental.pallas import tpu_sc as plsc

NC = 2
NS = 16
L = 16
NW = NC * NS
CH = 128
NB = 2


def _rsqrt(x):
    i = lax.bitcast_convert_type(x, jnp.int32)
    i = jnp.int32(0x5F3759DF) - (i >> 1)
    y = lax.bitcast_convert_type(i, jnp.float32)
    y = y * (1.5 - 0.5 * x * y * y)
    y = y * (1.5 - 0.5 * x * y * y)
    return y


def _cross(a, b):
    return (a[1] * b[2] - a[2] * b[1],
            a[2] * b[0] - a[0] * b[2],
            a[0] * b[1] - a[1] * b[0])


def _norm3(v):
    r = _rsqrt(v[0] * v[0] + v[1] * v[1] + v[2] * v[2])
    return v[0] * r, v[1] * r, v[2] * r


def _frame(a, b, c):
    d = (b[0] - a[0], b[1] - a[1], b[2] - a[2])
    e = (c[0] - a[0], c[1] - a[1], c[2] - a[2])
    n = _cross(d, e)
    X = _norm3(_cross(d, n))
    Z = _norm3(d)
    Y = _cross(Z, X)
    return X, Y, Z


def _quat_w(ca, cb, cc, ma, mb, mc):
    Xc, Yc, Zc = _frame(ca, cb, cc)
    Xd, Yd, Zd = _frame(ma, mb, mc)

    def m(i, j):
        return Xd[i] * Xc[j] + Yd[i] * Yc[j] + Zd[i] * Zc[j]

    m00, m01, m02 = m(0, 0), m(0, 1), m(0, 2)
    m10, m11, m12 = m(1, 0), m(1, 1), m(1, 2)
    m20, m21, m22 = m(2, 0), m(2, 1), m(2, 2)

    def sqrtpos(t):
        pos = t > 0.0
        return jnp.where(pos, t * _rsqrt(jnp.where(pos, t, 1.0)), 0.0)

    q0 = sqrtpos(1.0 + m00 + m11 + m22)
    q1 = sqrtpos(1.0 + m00 - m11 - m22)
    q2 = sqrtpos(1.0 - m00 + m11 - m22)
    q3 = sqrtpos(1.0 - m00 - m11 + m22)

    r0 = (q0 * q0, m21 - m12, m02 - m20, m10 - m01)
    r1 = (m21 - m12, q1 * q1, m10 + m01, m02 + m20)
    r2 = (m02 - m20, m10 + m01, q2 * q2, m12 + m21)
    r3 = (m10 - m01, m20 + m02, m21 + m12, q3 * q3)

    d0 = 2.0 * jnp.maximum(q0, 0.1)
    d1 = 2.0 * jnp.maximum(q1, 0.1)
    d2 = 2.0 * jnp.maximum(q2, 0.1)
    d3 = 2.0 * jnp.maximum(q3, 0.1)

    qm = jnp.maximum(jnp.maximum(q0, q1), jnp.maximum(q2, q3))
    s0, s1, s2 = q0 == qm, q1 == qm, q2 == qm

    u = (cc[0] - cb[0], cc[1] - cb[1], cc[2] - cb[2])
    v = (ca[0] - cb[0], ca[1] - cb[1], ca[2] - cb[2])
    f = _cross(u, v)
    ss = f[0] * f[0] + f[1] * f[1] + f[2] * f[2]
    area = 0.5 * ss * _rsqrt(jnp.maximum(ss, 1e-38))

    out = []
    for k in range(4):
        q = jnp.where(s0, r0[k] / d0,
            jnp.where(s1, r1[k] / d1,
            jnp.where(s2, r2[k] / d2, r3[k] / d3)))
        out.append(area * q)
    return out


def _build_face_kernel(Vp, FW, acc_tile):
    mesh = plsc.VectorSubcoreMesh(core_axis_name="c", subcore_axis_name="s",
                                  num_cores=NC)

    @pl.kernel(
        out_type=jax.ShapeDtypeStruct((NC * 4 * Vp,), jnp.float32),
        mesh=mesh,
        scratch_types=(
            [pltpu.VMEM((CH,), jnp.int32) for _ in range(3 * NB)]
            + [pltpu.VMEM((CH,), jnp.float32) for _ in range(18 * NB)]
            + [pltpu.VMEM((CH,), jnp.float32) for _ in range(4)]
            + [pltpu.VMEM((acc_tile,), jnp.float32)]
            + [pltpu.VMEM_SHARED((Vp,), jnp.float32) for _ in range(4)]
            + [pltpu.SemaphoreType.DMA for _ in range(NB)]
        ),
    )
    def face_kernel(cx, cy, cz, mx, my, mz, f0, f1, f2, out_hbm, *scr):
        ivs = [scr[3 * b:3 * b + 3] for b in range(NB)]
        o = 3 * NB
        gbs = [scr[o + 18 * b:o + 18 * b + 18] for b in range(NB)]
        o += 18 * NB
        wvs = scr[o:o + 4]
        stage = scr[o + 4]
        accs = scr[o + 5:o + 9]
        sems = scr[o + 9:o + 9 + NB]

        c = lax.axis_index("c")
        s = lax.axis_index("s")
        wid = s * NC + c
        tables = (cx, cy, cz, mx, my, mz)
        fidx = (f0, f1, f2)

        def zero_grp(g, carry):
            stage[pl.ds(pl.multiple_of(g * L, 8), L)] = jnp.zeros(
                (L,), jnp.float32)
            return carry

        lax.fori_loop(0, acc_tile // L, zero_grp, 0)
        zb = pl.multiple_of(s * acc_tile, 8)
        for k in range(4):
            pltpu.sync_copy(stage, accs[k].at[pl.ds(zb, acc_tile)])
        plsc.subcore_barrier()

        def load_idx(iv, chunk):
            base = pl.multiple_of(wid * FW + chunk * CH, 8)
            for ci in range(3):
                pltpu.sync_copy(fidx[ci].at[pl.ds(base, CH)], iv[ci])

        def issue_gathers(iv, gb, sem):
            for ti in range(6):
                for ci in range(3):
                    pltpu.async_copy(tables[ti].at[iv[ci]],
                                     gb[ti * 3 + ci], sem)

        def drain_gathers(iv, gb, sem):
            for ti in range(6):
                for ci in range(3):
                    pltpu.make_async_copy(tables[ti].at[iv[ci]],
                                          gb[ti * 3 + ci], sem).wait()

        for b in range(NB):
            load_idx(ivs[b], b)
            issue_gathers(ivs[b], gbs[b], sems[b])

        def pair(t, carry):
            for b in range(NB):
                iv, gb, sem = ivs[b], gbs[b], sems[b]
                chunk = NB * t + b
                drain_gathers(iv, gb, sem)

                def group(g, carry2):
                    sl = pl.ds(pl.multiple_of(g * L, 8), L)
                    w = _quat_w(
                        (gb[0][sl], gb[3][sl], gb[6][sl]),
                        (gb[1][sl], gb[4][sl], gb[7][sl]),
                        (gb[2][sl], gb[5][sl], gb[8][sl]),
                        (gb[9][sl], gb[12][sl], gb[15][sl]),
                        (gb[10][sl], gb[13][sl], gb[16][sl]),
                        (gb[11][sl], gb[14][sl], gb[17][sl]))
                    for k in range(4):
                        wvs[k][sl] = w[k]
                    return carry2

                lax.fori_loop(0, CH // L, group, 0)

                for ci in range(3):
                    for k in range(4):
                        pltpu.sync_copy(wvs[k], accs[k].at[iv[ci]], add=True)

                load_idx(iv, chunk + NB)
                issue_gathers(iv, gb, sem)
            return carry

        lax.fori_loop(0, FW // (NB * CH), pair, 0)
        for b in range(NB):
            drain_gathers(ivs[b], gbs[b], sems[b])
        plsc.subcore_barrier()

        for k in range(4):
            ob = pl.multiple_of((c * 4 + k) * Vp + zb, 8)
            pltpu.sync_copy(accs[k].at[pl.ds(zb, acc_tile)], stage)
            pltpu.sync_copy(stage, out_hbm.at[pl.ds(ob, acc_tile)])

    return face_kernel


def _build_combine_kernel(Vp, VW):
    mesh = plsc.VectorSubcoreMesh(core_axis_name="c", subcore_axis_name="s",
                                  num_cores=NC)

    @pl.kernel(
        out_type=jax.ShapeDtypeStruct((4 * Vp,), jnp.float32),
        mesh=mesh,
        scratch_types=(
            [pltpu.VMEM((VW,), jnp.float32) for _ in range(12)]
        ),
    )
    def combine_kernel(parts_hbm, out_hbm,
                       p0, p1, p2, p3, q0, q1, q2, q3, o0, o1, o2, o3):
        c = lax.axis_index("c")
        s = lax.axis_index("s")
        wid = s * NC + c
        base = pl.multiple_of(wid * VW, 8)
        pv = (p0, p1, p2, p3)
        qv = (q0, q1, q2, q3)
        ov = (o0, o1, o2, o3)
        for k in range(4):
            pb = pl.multiple_of(k * Vp + base, 8)
            qb = pl.multiple_of((4 + k) * Vp + base, 8)
            pltpu.sync_copy(parts_hbm.at[pl.ds(pb, VW)], pv[k])
            pltpu.sync_copy(parts_hbm.at[pl.ds(qb, VW)], qv[k])

        def grp(g, carry):
            sl = pl.ds(pl.multiple_of(g * L, 8), L)
            sv = [pv[k][sl] + qv[k][sl] for k in range(4)]
            ss = sv[0] * sv[0] + sv[1] * sv[1] + sv[2] * sv[2] + sv[3] * sv[3]
            r = _rsqrt(jnp.maximum(ss, 1e-38))
            nrm = ss * r
            inv = jnp.where(nrm > 1e-6, r, 1e6)
            for k in range(4):
                ov[k][sl] = sv[k] * inv
            return carry

        lax.fori_loop(0, VW // L, grp, 0)
        for k in range(4):
            ob = pl.multiple_of(k * Vp + base, 8)
            pltpu.sync_copy(ov[k], out_hbm.at[pl.ds(ob, VW)])

    return combine_kernel


def kernel(mesh_verts, cano_verts, cano_faces):
    V = cano_verts.shape[0]
    F = cano_faces.shape[0]

    FW = -(-F // (NW * NB * CH)) * NB * CH
    F_pad = NW * FW + NB * CH
    VW = -(-V // (NW * L)) * L
    Vp = NW * VW
    acc_tile = Vp // NS

    cano = cano_verts.astype(jnp.float32)
    dmesh = mesh_verts.astype(jnp.float32)
    faces = jnp.pad(cano_faces.astype(jnp.int32), ((0, F_pad - F), (0, 0)))

    parts = _build_face_kernel(Vp, FW, acc_tile)(
        cano[:, 0], cano[:, 1], cano[:, 2],
        dmesh[:, 0], dmesh[:, 1], dmesh[:, 2],
        faces[:, 0], faces[:, 1], faces[:, 2])
    out = _build_combine_kernel(Vp, VW)(parts)
    return out.reshape(4, Vp).T[:V]
